# baseline (speedup 1.0000x reference)
import functools

import jax
import jax.numpy as jnp
from jax import lax
from jax.experimental import pallas as pl
from jax.experimental.pallas import tpu as pltpu
from jax.experimental.pallas import tpu_sc as plsc

N_NODES = 10000
N_EDGES = 320000
D_IN = 128
D_HID = 128
D_OUT = 16

NC = 2
NS = 16
CH = 128
NB = 80
E_PAD = NC * NS * NB * CH
NBF = NB * NC
N_ACC = 10496
RPT = N_ACC // NS
BR = 400
NBLK = N_NODES // BR

_MESH = plsc.VectorSubcoreMesh(core_axis_name="c", subcore_axis_name="s")
_L1_FEATURE_SPLIT = False


@functools.partial(
    pl.kernel,
    out_type=jax.ShapeDtypeStruct((NC, N_ACC, 16), jnp.float32),
    mesh=_MESH,
    scratch_types=[
        pltpu.VMEM((NB, CH), jnp.int32),
        pltpu.VMEM((CH, 16), jnp.float32),
        pltpu.VMEM_SHARED((N_ACC, 16), jnp.float32),
        pltpu.SemaphoreType.DMA,
    ],
)
def _sc_degree(col_hbm, z_hbm, out_hbm, col_v, ones_v, acc_sp, sem):
    c = lax.axis_index("c")
    s = lax.axis_index("s")
    pltpu.sync_copy(z_hbm.at[pl.ds(pl.multiple_of(s * RPT, 8), RPT)], acc_sp.at[pl.ds(pl.multiple_of(s * RPT, 8), RPT)])
    pltpu.sync_copy(col_hbm.at[c, s], col_v)

    def fill(i, carry):
        ones_v[i, :] = jnp.ones((16,), jnp.float32)
        return carry

    lax.fori_loop(0, CH, fill, 0)
    plsc.subcore_barrier()

    def body(j, carry):
        pltpu.async_copy(ones_v, acc_sp.at[col_v.at[j]], sem, add=True)
        return carry

    lax.fori_loop(0, NB, body, 0)

    def drain(j, carry):
        pltpu.make_async_copy(ones_v, acc_sp.at[col_v.at[j]], sem).wait()
        return carry

    lax.fori_loop(0, NB, drain, 0)
    plsc.subcore_barrier()
    pltpu.sync_copy(acc_sp.at[pl.ds(pl.multiple_of(s * RPT, 8), RPT)], out_hbm.at[c, pl.ds(pl.multiple_of(s * RPT, 8), RPT)])


def _make_sc_scatter(w, grp):
    @functools.partial(
        pl.kernel,
        out_type=jax.ShapeDtypeStruct((NC, N_ACC, w), jnp.float32),
        mesh=_MESH,
        compiler_params=pltpu.CompilerParams(use_tc_tiling_on_sc=(w == D_HID)),
        scratch_types=[
            pltpu.VMEM((NB, CH), jnp.int32),
            pltpu.VMEM((NB, CH), jnp.int32),
            pltpu.VMEM((2 * grp, CH, w), jnp.float32),
            pltpu.VMEM_SHARED((N_ACC, w), jnp.float32),
            pltpu.SemaphoreType.DMA,
            pltpu.SemaphoreType.DMA,
        ],
    )
    def scat(table_hbm, row_hbm, col_hbm, z_hbm, out_hbm,
             row_v, col_v, msg_v, acc_sp, gsem, ssem):
        c = lax.axis_index("c")
        s = lax.axis_index("s")
        pltpu.sync_copy(z_hbm.at[pl.ds(pl.multiple_of(s * RPT, 8), RPT)], acc_sp.at[pl.ds(pl.multiple_of(s * RPT, 8), RPT)])
        pltpu.sync_copy(row_hbm.at[c, s], row_v)
        pltpu.sync_copy(col_hbm.at[c, s], col_v)
        plsc.subcore_barrier()

        def body(g, carry):
            bs = lax.rem(g, 2) * grp
            base = g * grp

            @pl.when(g >= 2)
            def _():
                for k in range(grp):
                    pltpu.make_async_copy(
                        msg_v.at[bs + k], acc_sp.at[col_v.at[base + k]], ssem).wait()

            gd = [
                pltpu.async_copy(table_hbm.at[row_v.at[base + k]], msg_v.at[bs + k], gsem)
                for k in range(grp)
            ]
            for k in range(grp):
                gd[k].wait()
                pltpu.async_copy(
                    msg_v.at[bs + k], acc_sp.at[col_v.at[base + k]], ssem, add=True)
            return carry

        ng = NB // grp
        lax.fori_loop(0, ng, body, 0)
        for k in range(2 * grp):
            pltpu.make_async_copy(msg_v.at[0], acc_sp.at[col_v.at[0]], ssem).wait()
        plsc.subcore_barrier()
        pltpu.sync_copy(acc_sp.at[pl.ds(pl.multiple_of(s * RPT, 8), RPT)], out_hbm.at[c, pl.ds(pl.multiple_of(s * RPT, 8), RPT)])

    return scat


_sc_scatter16 = _make_sc_scatter(D_OUT, 8)


def _make_sc_scatter_fs(w, grp, edge_split=False):
    nch = NB if edge_split else NBF
    ng = nch // grp

    @functools.partial(
        pl.kernel,
        out_type=jax.ShapeDtypeStruct((NC, N_ACC, w), jnp.float32),
        mesh=_MESH,
        compiler_params=pltpu.CompilerParams(use_tc_tiling_on_sc=False),
        scratch_types=[
            pltpu.VMEM((2 * grp, CH), jnp.int32),
            pltpu.VMEM((2 * grp, CH), jnp.int32),
            pltpu.VMEM((2 * grp, CH, w), jnp.float32),
            pltpu.VMEM_SHARED((N_ACC, w), jnp.float32),
            pltpu.SemaphoreType.DMA,
            pltpu.SemaphoreType.DMA,
            pltpu.SemaphoreType.DMA,
        ],
    )
    def scat(table_hbm, row_hbm, col_hbm, z_hbm, out_hbm,
             row_v, col_v, msg_v, acc_sp, gsem, ssem, isem):
        c = lax.axis_index("c")
        s = lax.axis_index("s")

        def row_src(g0):
            return row_hbm.at[c, s, pl.ds(g0, grp)]

        def col_src(g0):
            if edge_split:
                return col_hbm.at[c, s, pl.ds(g0, grp)]
            return col_hbm.at[s, pl.ds(g0, grp)]

        pltpu.sync_copy(z_hbm.at[pl.ds(pl.multiple_of(s * RPT, 8), RPT)], acc_sp.at[pl.ds(pl.multiple_of(s * RPT, 8), RPT)])
        pltpu.async_copy(row_src(0), row_v.at[pl.ds(0, grp)], isem)
        pltpu.async_copy(col_src(0), col_v.at[pl.ds(0, grp)], isem)
        plsc.subcore_barrier()

        def body(g, carry):
            bs = lax.rem(g, 2) * grp
            pltpu.make_async_copy(row_src(0), row_v.at[pl.ds(0, grp)], isem).wait()
            pltpu.make_async_copy(col_src(0), col_v.at[pl.ds(0, grp)], isem).wait()

            gd = [
                pltpu.async_copy(table_hbm.at[row_v.at[bs + k]], msg_v.at[bs + k], gsem)
                for k in range(grp)
            ]

            @pl.when(g >= 1)
            def _():
                for k in range(grp):
                    pltpu.make_async_copy(
                        msg_v.at[bs + k], acc_sp.at[col_v.at[bs + k]], ssem).wait()

            for k in range(grp):
                gd[k].wait()
                pltpu.async_copy(
                    msg_v.at[bs + k], acc_sp.at[col_v.at[bs + k]], ssem, add=True)

            @pl.when(g + 1 < ng)
            def _():
                nb = (1 - lax.rem(g, 2)) * grp
                nxt = (g + 1) * grp
                pltpu.async_copy(row_src(nxt), row_v.at[pl.ds(nb, grp)], isem)
                pltpu.async_copy(col_src(nxt), col_v.at[pl.ds(nb, grp)], isem)
            return carry

        lax.fori_loop(0, ng, body, 0)
        for k in range(grp):
            pltpu.make_async_copy(msg_v.at[0], acc_sp.at[col_v.at[0]], ssem).wait()
        plsc.subcore_barrier()
        pltpu.sync_copy(acc_sp.at[pl.ds(pl.multiple_of(s * RPT, 8), RPT)], out_hbm.at[c, pl.ds(pl.multiple_of(s * RPT, 8), RPT)])

    return scat


_sc_scatter64 = _make_sc_scatter_fs(D_HID // 2, 4)
_sc_scatter128 = _make_sc_scatter_fs(D_HID, 1, edge_split=True)


def _tc_matmul1(x, w1):
    def body(x_ref, w_ref, h_ref):
        h_ref[...] = jnp.dot(x_ref[...], w_ref[...], preferred_element_type=jnp.float32)

    return pl.pallas_call(
        body,
        grid=(NBLK,),
        in_specs=[
            pl.BlockSpec((BR, D_IN), lambda i: (i, 0)),
            pl.BlockSpec((D_IN, D_HID), lambda i: (0, 0)),
        ],
        out_specs=pl.BlockSpec((BR, D_HID), lambda i: (i, 0)),
        out_shape=jax.ShapeDtypeStruct((N_NODES, D_HID), jnp.float32),
    )(x, w1)


def _tc_scale1(h1, deg2):
    def body(h_ref, dega_ref, degb_ref, h_out, dinv_ref):
        deg = dega_ref[0, :, :1] + degb_ref[0, :, :1] + 1.0
        dinv = lax.rsqrt(deg)
        h_out[...] = h_ref[...] * dinv
        dinv_ref[...] = jnp.broadcast_to(dinv, (BR, 16))

    return pl.pallas_call(
        body,
        grid=(NBLK,),
        in_specs=[
            pl.BlockSpec((BR, D_HID), lambda i: (i, 0)),
            pl.BlockSpec((1, BR, 16), lambda i: (0, i, 0)),
            pl.BlockSpec((1, BR, 16), lambda i: (1, i, 0)),
        ],
        out_specs=[
            pl.BlockSpec((BR, D_HID), lambda i: (i, 0)),
            pl.BlockSpec((BR, 16), lambda i: (i, 0)),
        ],
        out_shape=[
            jax.ShapeDtypeStruct((N_NODES, D_HID), jnp.float32),
            jax.ShapeDtypeStruct((N_NODES, 16), jnp.float32),
        ],
    )(h1, deg2, deg2)


def _tc_layer2(agg1, h1s, dinv16, b1, w2):
    aw = agg1.shape[-1]

    def body(agg_ref, h1_ref, dinv_ref, b1_ref, w2_ref, out_ref):
        dinv = dinv_ref[:, :1]
        if aw == D_HID:
            pres = agg_ref[0] + agg_ref[1] + h1_ref[...]
        else:
            pres = jnp.concatenate([agg_ref[0], agg_ref[1]], axis=-1) + h1_ref[...]
        h = jnp.maximum(pres * dinv + b1_ref[...], 0.0)
        out_ref[...] = jnp.dot(h, w2_ref[...], preferred_element_type=jnp.float32) * dinv

    return pl.pallas_call(
        body,
        grid=(NBLK,),
        in_specs=[
            pl.BlockSpec((NC, BR, aw), lambda i: (0, i, 0)),
            pl.BlockSpec((BR, D_HID), lambda i: (i, 0)),
            pl.BlockSpec((BR, 16), lambda i: (i, 0)),
            pl.BlockSpec((1, D_HID), lambda i: (0, 0)),
            pl.BlockSpec((D_HID, D_OUT), lambda i: (0, 0)),
        ],
        out_specs=pl.BlockSpec((BR, D_OUT), lambda i: (i, 0)),
        out_shape=jax.ShapeDtypeStruct((N_NODES, D_OUT), jnp.float32),
    )(agg1, h1s, dinv16, b1.reshape(1, D_HID), w2)


def _tc_final(agg2, h2s, dinv16, b2):
    def body(agg_ref, h2_ref, dinv_ref, b2_ref, out_ref):
        pres = agg_ref[0] + agg_ref[1] + h2_ref[...]
        out_ref[...] = pres * dinv_ref[:, :1] + b2_ref[...]

    return pl.pallas_call(
        body,
        grid=(NBLK,),
        in_specs=[
            pl.BlockSpec((NC, BR, D_OUT), lambda i: (0, i, 0)),
            pl.BlockSpec((BR, D_OUT), lambda i: (i, 0)),
            pl.BlockSpec((BR, 16), lambda i: (i, 0)),
            pl.BlockSpec((1, D_OUT), lambda i: (0, 0)),
        ],
        out_specs=pl.BlockSpec((BR, D_OUT), lambda i: (i, 0)),
        out_shape=jax.ShapeDtypeStruct((N_NODES, D_OUT), jnp.float32),
    )(agg2, h2s, dinv16, b2.reshape(1, D_OUT))


def kernel(x, edge_index, W1, b1, W2, b2):
    row = edge_index[0].astype(jnp.int32)
    col = edge_index[1].astype(jnp.int32)
    npad = E_PAD - N_EDGES
    row_p = jnp.concatenate([row, jnp.zeros((npad,), jnp.int32)])
    col_p = jnp.concatenate([col, jnp.full((npad,), N_NODES, jnp.int32)])
    row_r = row_p
    col_r = col_p
    row_fs = jnp.stack([2 * row_r, 2 * row_r + 1]).reshape(NC, NS, NBF, CH)
    col_fs = col_r.reshape(NS, NBF, CH)
    row_r = row_r.reshape(NC, NS, NB, CH)
    col_r = col_r.reshape(NC, NS, NB, CH)

    z16 = jnp.zeros((N_ACC, 16), jnp.float32)

    deg2 = _sc_degree(col_r, z16)
    h1 = _tc_matmul1(x, W1)
    h1s, dinv16 = _tc_scale1(h1, deg2)
    if _L1_FEATURE_SPLIT:
        z64 = jnp.zeros((N_ACC, D_HID // 2), jnp.float32)
        table1 = h1s.reshape(2 * N_NODES, D_HID // 2)
        agg1 = _sc_scatter64(table1, row_fs, col_fs, z64)
    else:
        z128 = jnp.zeros((N_ACC, D_HID), jnp.float32)
        agg1 = _sc_scatter128(h1s, row_r, col_r, z128)
    h2s = _tc_layer2(agg1, h1s, dinv16, b1, W2)
    agg2 = _sc_scatter16(h2s, row_r, col_r, z16)
    return _tc_final(agg2, h2s, dinv16, b2)


# speedup vs baseline: 2.4102x; 2.4102x over previous
import functools

import jax
import jax.numpy as jnp
from jax import lax
from jax.experimental import pallas as pl
from jax.experimental.pallas import tpu as pltpu
from jax.experimental.pallas import tpu_sc as plsc

N_NODES = 10000
N_EDGES = 320000
D_IN = 128
D_HID = 128
D_OUT = 16

NC = 2
NS = 16
CH = 128
NB = 80
E_PAD = NC * NS * NB * CH
NBF = NB * NC
N_ACC = 10496
RPT = N_ACC // NS
BR = 400
NBLK = N_NODES // BR

_MESH = plsc.VectorSubcoreMesh(core_axis_name="c", subcore_axis_name="s")
_L1_FEATURE_SPLIT = True


@functools.partial(
    pl.kernel,
    out_type=jax.ShapeDtypeStruct((NC, N_ACC, 16), jnp.float32),
    mesh=_MESH,
    scratch_types=[
        pltpu.VMEM((NB, CH), jnp.int32),
        pltpu.VMEM((CH, 16), jnp.float32),
        pltpu.VMEM_SHARED((N_ACC, 16), jnp.float32),
        pltpu.SemaphoreType.DMA,
    ],
)
def _sc_degree(col_hbm, z_hbm, out_hbm, col_v, ones_v, acc_sp, sem):
    c = lax.axis_index("c")
    s = lax.axis_index("s")
    pltpu.sync_copy(z_hbm.at[pl.ds(pl.multiple_of(s * RPT, 8), RPT)], acc_sp.at[pl.ds(pl.multiple_of(s * RPT, 8), RPT)])
    pltpu.sync_copy(col_hbm.at[c, s], col_v)

    def fill(i, carry):
        ones_v[i, :] = jnp.ones((16,), jnp.float32)
        return carry

    lax.fori_loop(0, CH, fill, 0)
    plsc.subcore_barrier()

    def body(j, carry):
        pltpu.async_copy(ones_v, acc_sp.at[col_v.at[j]], sem, add=True)
        return carry

    lax.fori_loop(0, NB, body, 0)

    def drain(j, carry):
        pltpu.make_async_copy(ones_v, acc_sp.at[col_v.at[j]], sem).wait()
        return carry

    lax.fori_loop(0, NB, drain, 0)
    plsc.subcore_barrier()
    pltpu.sync_copy(acc_sp.at[pl.ds(pl.multiple_of(s * RPT, 8), RPT)], out_hbm.at[c, pl.ds(pl.multiple_of(s * RPT, 8), RPT)])


def _make_sc_scatter(w, grp):
    @functools.partial(
        pl.kernel,
        out_type=jax.ShapeDtypeStruct((NC, N_ACC, w), jnp.float32),
        mesh=_MESH,
        compiler_params=pltpu.CompilerParams(use_tc_tiling_on_sc=(w == D_HID)),
        scratch_types=[
            pltpu.VMEM((NB, CH), jnp.int32),
            pltpu.VMEM((NB, CH), jnp.int32),
            pltpu.VMEM((2 * grp, CH, w), jnp.float32),
            pltpu.VMEM_SHARED((N_ACC, w), jnp.float32),
            pltpu.SemaphoreType.DMA,
            pltpu.SemaphoreType.DMA,
        ],
    )
    def scat(table_hbm, row_hbm, col_hbm, z_hbm, out_hbm,
             row_v, col_v, msg_v, acc_sp, gsem, ssem):
        c = lax.axis_index("c")
        s = lax.axis_index("s")
        pltpu.sync_copy(z_hbm.at[pl.ds(pl.multiple_of(s * RPT, 8), RPT)], acc_sp.at[pl.ds(pl.multiple_of(s * RPT, 8), RPT)])
        pltpu.sync_copy(row_hbm.at[c, s], row_v)
        pltpu.sync_copy(col_hbm.at[c, s], col_v)
        plsc.subcore_barrier()

        def body(g, carry):
            bs = lax.rem(g, 2) * grp
            base = g * grp

            @pl.when(g >= 2)
            def _():
                for k in range(grp):
                    pltpu.make_async_copy(
                        msg_v.at[bs + k], acc_sp.at[col_v.at[base + k]], ssem).wait()

            gd = [
                pltpu.async_copy(table_hbm.at[row_v.at[base + k]], msg_v.at[bs + k], gsem)
                for k in range(grp)
            ]
            for k in range(grp):
                gd[k].wait()
                pltpu.async_copy(
                    msg_v.at[bs + k], acc_sp.at[col_v.at[base + k]], ssem, add=True)
            return carry

        ng = NB // grp
        lax.fori_loop(0, ng, body, 0)
        for k in range(2 * grp):
            pltpu.make_async_copy(msg_v.at[0], acc_sp.at[col_v.at[0]], ssem).wait()
        plsc.subcore_barrier()
        pltpu.sync_copy(acc_sp.at[pl.ds(pl.multiple_of(s * RPT, 8), RPT)], out_hbm.at[c, pl.ds(pl.multiple_of(s * RPT, 8), RPT)])

    return scat


_sc_scatter16 = _make_sc_scatter(D_OUT, 8)


def _make_sc_scatter_sp(w, grp, edge_split=False):
    nch = NB if edge_split else NBF
    ng = nch // grp
    nst = N_NODES // NS

    @functools.partial(
        pl.kernel,
        out_type=jax.ShapeDtypeStruct((NC, N_ACC, w), jnp.float32),
        mesh=_MESH,
        compiler_params=pltpu.CompilerParams(use_tc_tiling_on_sc=False),
        scratch_types=[
            pltpu.VMEM((2 * grp, CH), jnp.int32),
            pltpu.VMEM((2 * grp, CH), jnp.int32),
            pltpu.VMEM((2 * grp, CH, w), jnp.float32),
            pltpu.VMEM_SHARED((N_NODES, w), jnp.float32),
            pltpu.VMEM_SHARED((N_ACC, w), jnp.float32),
            pltpu.SemaphoreType.DMA,
            pltpu.SemaphoreType.DMA,
            pltpu.SemaphoreType.DMA,
        ],
    )
    def scat(table_hbm, row_hbm, col_hbm, z_hbm, out_hbm,
             row_v, col_v, msg_v, tbl_sp, acc_sp, gsem, ssem, isem):
        c = lax.axis_index("c")
        s = lax.axis_index("s")

        def row_src(g0):
            if edge_split:
                return row_hbm.at[c, s, pl.ds(g0, grp)]
            return row_hbm.at[s, pl.ds(g0, grp)]

        def col_src(g0):
            if edge_split:
                return col_hbm.at[c, s, pl.ds(g0, grp)]
            return col_hbm.at[s, pl.ds(g0, grp)]

        pltpu.sync_copy(z_hbm.at[pl.ds(pl.multiple_of(s * RPT, 8), RPT)], acc_sp.at[pl.ds(pl.multiple_of(s * RPT, 8), RPT)])
        if edge_split:
            pltpu.sync_copy(table_hbm.at[pl.ds(s * nst, nst)], tbl_sp.at[pl.ds(s * nst, nst)])
        else:
            pltpu.sync_copy(table_hbm.at[c, pl.ds(s * nst, nst)], tbl_sp.at[pl.ds(s * nst, nst)])
        pltpu.async_copy(row_src(0), row_v.at[pl.ds(0, grp)], isem)
        pltpu.async_copy(col_src(0), col_v.at[pl.ds(0, grp)], isem)
        plsc.subcore_barrier()

        def body(g, carry):
            bs = lax.rem(g, 2) * grp
            pltpu.make_async_copy(row_src(0), row_v.at[pl.ds(0, grp)], isem).wait()
            pltpu.make_async_copy(col_src(0), col_v.at[pl.ds(0, grp)], isem).wait()

            gd = [
                pltpu.async_copy(tbl_sp.at[row_v.at[bs + k]], msg_v.at[bs + k], gsem)
                for k in range(grp)
            ]

            @pl.when(g >= 1)
            def _():
                for k in range(grp):
                    pltpu.make_async_copy(
                        msg_v.at[bs + k], acc_sp.at[col_v.at[bs + k]], ssem).wait()

            for k in range(grp):
                gd[k].wait()
                pltpu.async_copy(
                    msg_v.at[bs + k], acc_sp.at[col_v.at[bs + k]], ssem, add=True)

            @pl.when(g + 1 < ng)
            def _():
                nb = (1 - lax.rem(g, 2)) * grp
                nxt = (g + 1) * grp
                pltpu.async_copy(row_src(nxt), row_v.at[pl.ds(nb, grp)], isem)
                pltpu.async_copy(col_src(nxt), col_v.at[pl.ds(nb, grp)], isem)
            return carry

        lax.fori_loop(0, ng, body, 0)
        for k in range(grp):
            pltpu.make_async_copy(msg_v.at[0], acc_sp.at[col_v.at[0]], ssem).wait()
        plsc.subcore_barrier()
        pltpu.sync_copy(acc_sp.at[pl.ds(pl.multiple_of(s * RPT, 8), RPT)], out_hbm.at[c, pl.ds(pl.multiple_of(s * RPT, 8), RPT)])

    return scat


def _make_sc_scatter_fs(w, grp, edge_split=False):
    nch = NB if edge_split else NBF
    ng = nch // grp

    @functools.partial(
        pl.kernel,
        out_type=jax.ShapeDtypeStruct((NC, N_ACC, w), jnp.float32),
        mesh=_MESH,
        compiler_params=pltpu.CompilerParams(use_tc_tiling_on_sc=False),
        scratch_types=[
            pltpu.VMEM((2 * grp, CH), jnp.int32),
            pltpu.VMEM((2 * grp, CH), jnp.int32),
            pltpu.VMEM((2 * grp, CH, w), jnp.float32),
            pltpu.VMEM_SHARED((N_ACC, w), jnp.float32),
            pltpu.SemaphoreType.DMA,
            pltpu.SemaphoreType.DMA,
            pltpu.SemaphoreType.DMA,
        ],
    )
    def scat(table_hbm, row_hbm, col_hbm, z_hbm, out_hbm,
             row_v, col_v, msg_v, acc_sp, gsem, ssem, isem):
        c = lax.axis_index("c")
        s = lax.axis_index("s")

        def row_src(g0):
            return row_hbm.at[c, s, pl.ds(g0, grp)]

        def col_src(g0):
            if edge_split:
                return col_hbm.at[c, s, pl.ds(g0, grp)]
            return col_hbm.at[s, pl.ds(g0, grp)]

        pltpu.sync_copy(z_hbm.at[pl.ds(pl.multiple_of(s * RPT, 8), RPT)], acc_sp.at[pl.ds(pl.multiple_of(s * RPT, 8), RPT)])
        pltpu.async_copy(row_src(0), row_v.at[pl.ds(0, grp)], isem)
        pltpu.async_copy(col_src(0), col_v.at[pl.ds(0, grp)], isem)
        plsc.subcore_barrier()

        def body(g, carry):
            bs = lax.rem(g, 2) * grp
            pltpu.make_async_copy(row_src(0), row_v.at[pl.ds(0, grp)], isem).wait()
            pltpu.make_async_copy(col_src(0), col_v.at[pl.ds(0, grp)], isem).wait()

            gd = [
                pltpu.async_copy(table_hbm.at[row_v.at[bs + k]], msg_v.at[bs + k], gsem)
                for k in range(grp)
            ]

            @pl.when(g >= 1)
            def _():
                for k in range(grp):
                    pltpu.make_async_copy(
                        msg_v.at[bs + k], acc_sp.at[col_v.at[bs + k]], ssem).wait()

            for k in range(grp):
                gd[k].wait()
                pltpu.async_copy(
                    msg_v.at[bs + k], acc_sp.at[col_v.at[bs + k]], ssem, add=True)

            @pl.when(g + 1 < ng)
            def _():
                nb = (1 - lax.rem(g, 2)) * grp
                nxt = (g + 1) * grp
                pltpu.async_copy(row_src(nxt), row_v.at[pl.ds(nb, grp)], isem)
                pltpu.async_copy(col_src(nxt), col_v.at[pl.ds(nb, grp)], isem)
            return carry

        lax.fori_loop(0, ng, body, 0)
        for k in range(grp):
            pltpu.make_async_copy(msg_v.at[0], acc_sp.at[col_v.at[0]], ssem).wait()
        plsc.subcore_barrier()
        pltpu.sync_copy(acc_sp.at[pl.ds(pl.multiple_of(s * RPT, 8), RPT)], out_hbm.at[c, pl.ds(pl.multiple_of(s * RPT, 8), RPT)])

    return scat


_sc_scatter64 = _make_sc_scatter_fs(D_HID // 2, 4)
_sc_scatter128 = _make_sc_scatter_fs(D_HID, 1, edge_split=True)
_sc_scatter_sp64 = _make_sc_scatter_sp(D_HID // 2, 2)
_sc_scatter_sp16 = _make_sc_scatter_sp(D_OUT, 8, edge_split=True)


def _tc_matmul1(x, w1):
    def body(x_ref, w_ref, h_ref):
        h_ref[...] = jnp.dot(x_ref[...], w_ref[...], preferred_element_type=jnp.float32)

    return pl.pallas_call(
        body,
        grid=(NBLK,),
        in_specs=[
            pl.BlockSpec((BR, D_IN), lambda i: (i, 0)),
            pl.BlockSpec((D_IN, D_HID), lambda i: (0, 0)),
        ],
        out_specs=pl.BlockSpec((BR, D_HID), lambda i: (i, 0)),
        out_shape=jax.ShapeDtypeStruct((N_NODES, D_HID), jnp.float32),
    )(x, w1)


def _tc_scale1(h1, deg2):
    def body(h_ref, dega_ref, degb_ref, h_out, dinv_ref, tab_ref):
        deg = dega_ref[0, :, :1] + degb_ref[0, :, :1] + 1.0
        dinv = lax.rsqrt(deg)
        hs = h_ref[...] * dinv
        h_out[...] = hs
        dinv_ref[...] = jnp.broadcast_to(dinv, (BR, 16))
        tab_ref[0] = hs[:, : D_HID // 2]
        tab_ref[1] = hs[:, D_HID // 2 :]

    return pl.pallas_call(
        body,
        grid=(NBLK,),
        in_specs=[
            pl.BlockSpec((BR, D_HID), lambda i: (i, 0)),
            pl.BlockSpec((1, BR, 16), lambda i: (0, i, 0)),
            pl.BlockSpec((1, BR, 16), lambda i: (1, i, 0)),
        ],
        out_specs=[
            pl.BlockSpec((BR, D_HID), lambda i: (i, 0)),
            pl.BlockSpec((BR, 16), lambda i: (i, 0)),
            pl.BlockSpec((NC, BR, D_HID // 2), lambda i: (0, i, 0)),
        ],
        out_shape=[
            jax.ShapeDtypeStruct((N_NODES, D_HID), jnp.float32),
            jax.ShapeDtypeStruct((N_NODES, 16), jnp.float32),
            jax.ShapeDtypeStruct((NC, N_NODES, D_HID // 2), jnp.float32),
        ],
    )(h1, deg2, deg2)


def _tc_layer2(agg1, h1s, dinv16, b1, w2):
    aw = agg1.shape[-1]

    def body(agg_ref, h1_ref, dinv_ref, b1_ref, w2_ref, out_ref):
        dinv = dinv_ref[:, :1]
        if aw == D_HID:
            pres = agg_ref[0] + agg_ref[1] + h1_ref[...]
        else:
            pres = jnp.concatenate([agg_ref[0], agg_ref[1]], axis=-1) + h1_ref[...]
        h = jnp.maximum(pres * dinv + b1_ref[...], 0.0)
        out_ref[...] = jnp.dot(h, w2_ref[...], preferred_element_type=jnp.float32) * dinv

    return pl.pallas_call(
        body,
        grid=(NBLK,),
        in_specs=[
            pl.BlockSpec((NC, BR, aw), lambda i: (0, i, 0)),
            pl.BlockSpec((BR, D_HID), lambda i: (i, 0)),
            pl.BlockSpec((BR, 16), lambda i: (i, 0)),
            pl.BlockSpec((1, D_HID), lambda i: (0, 0)),
            pl.BlockSpec((D_HID, D_OUT), lambda i: (0, 0)),
        ],
        out_specs=pl.BlockSpec((BR, D_OUT), lambda i: (i, 0)),
        out_shape=jax.ShapeDtypeStruct((N_NODES, D_OUT), jnp.float32),
    )(agg1, h1s, dinv16, b1.reshape(1, D_HID), w2)


def _tc_final(agg2, h2s, dinv16, b2):
    def body(agg_ref, h2_ref, dinv_ref, b2_ref, out_ref):
        pres = agg_ref[0] + agg_ref[1] + h2_ref[...]
        out_ref[...] = pres * dinv_ref[:, :1] + b2_ref[...]

    return pl.pallas_call(
        body,
        grid=(NBLK,),
        in_specs=[
            pl.BlockSpec((NC, BR, D_OUT), lambda i: (0, i, 0)),
            pl.BlockSpec((BR, D_OUT), lambda i: (i, 0)),
            pl.BlockSpec((BR, 16), lambda i: (i, 0)),
            pl.BlockSpec((1, D_OUT), lambda i: (0, 0)),
        ],
        out_specs=pl.BlockSpec((BR, D_OUT), lambda i: (i, 0)),
        out_shape=jax.ShapeDtypeStruct((N_NODES, D_OUT), jnp.float32),
    )(agg2, h2s, dinv16, b2.reshape(1, D_OUT))


def kernel(x, edge_index, W1, b1, W2, b2):
    row = edge_index[0].astype(jnp.int32)
    col = edge_index[1].astype(jnp.int32)
    npad = E_PAD - N_EDGES
    row_p = jnp.concatenate([row, jnp.zeros((npad,), jnp.int32)])
    col_p = jnp.concatenate([col, jnp.full((npad,), N_NODES, jnp.int32)])
    row_r = row_p
    col_r = col_p
    row_fs = row_p.reshape(NS, NBF, CH)
    col_fs = col_p.reshape(NS, NBF, CH)
    row_r = row_r.reshape(NC, NS, NB, CH)
    col_r = col_r.reshape(NC, NS, NB, CH)

    z16 = jnp.zeros((N_ACC, 16), jnp.float32)

    deg2 = _sc_degree(col_r, z16)
    h1 = _tc_matmul1(x, W1)
    h1s, dinv16, tab2 = _tc_scale1(h1, deg2)
    z64 = jnp.zeros((N_ACC, D_HID // 2), jnp.float32)
    agg1 = _sc_scatter_sp64(tab2, row_fs, col_fs, z64)
    h2s = _tc_layer2(agg1, h1s, dinv16, b1, W2)
    agg2 = _sc_scatter_sp16(h2s, row_r, col_r, z16)
    return _tc_final(agg2, h2s, dinv16, b2)
